# baseline (speedup 1.0000x reference)
import functools

import jax
import jax.numpy as jnp
from jax import lax
from jax.experimental import pallas as pl
from jax.experimental.pallas import tpu as pltpu
from jax.experimental.pallas import tpu_sc as plsc

HEADS = 8
DH = 16
LANES = 16
NCORES = 2
NSUB = 16
NW = NCORES * NSUB
CH = 40
TRP = 104


def _lane_bcast(v, zeros16, lane):
    return v.at[zeros16 + lane].get(mode="promise_in_bounds")


def _proj_body(xa_ref, xp_ref, wa_ref, wp_ref, ba_ref, bp_ref,
               asrc_w_ref, adst_w_ref, zp_ref, esrc_ref, edst_ref):
    zp = jnp.dot(xp_ref[...], wp_ref[...], preferred_element_type=jnp.float32)
    zp = zp + bp_ref[...]
    za = jnp.dot(xa_ref[...], wa_ref[...], preferred_element_type=jnp.float32)
    za = za + ba_ref[...]
    zp_ref[...] = zp
    esrc_ref[...] = jnp.dot(zp, asrc_w_ref[...], preferred_element_type=jnp.float32)
    edst_ref[...] = jnp.dot(za, adst_w_ref[...], preferred_element_type=jnp.float32)


def _final_body(acc0_ref, acc1_ref, den0_ref, den1_ref, bexp_ref, wout_ref,
                bout_ref, out_ref):
    a = acc0_ref[...] + acc1_ref[...]
    dsum = den0_ref[...] + den1_ref[...] + 1e-16
    dinv = 1.0 / dsum
    expand = jnp.dot(dinv, bexp_ref[...], preferred_element_type=jnp.float32)
    h = jnp.maximum(a * expand, 0.0)
    out_ref[...] = jnp.dot(h, wout_ref[...], preferred_element_type=jnp.float32) + bout_ref[...]


def kernel(x_author, x_paper, edge_index_ap, edge_index_pa,
           W_author, b_author, W_paper, b_paper,
           att_src_ap, att_dst_ap, att_src_pa, att_dst_pa,
           Wk, bk, q, W_out, b_out):
    NA = x_author.shape[0]
    NPA = x_paper.shape[0]
    DIN = x_author.shape[1]
    OUT = W_author.shape[1]
    DOUT = W_out.shape[1]
    E = edge_index_pa.shape[1]

    assert OUT == HEADS * DH and E % NW == 0
    assert NA % LANES == 0
    epw = E // NW
    assert epw % CH == 0
    nchunk = epw // CH
    assert nchunk % 2 == 0
    rpt = NA // NSUB - (NA // NSUB) % LANES
    rpt_last = NA - (NSUB - 1) * rpt
    drpt = rpt // LANES
    DTOT = NA // LANES
    RTOT = NA + DTOT
    RTOT8 = -(-RTOT // 8) * 8
    zpt = (RTOT8 // NSUB) // 8 * 8
    zpt_last = RTOT8 - (NSUB - 1) * zpt
    ntr = rpt // TRP
    assert ntr * TRP == rpt and rpt_last - rpt <= TRP

    ridx = jnp.arange(OUT) // DH
    cidx = jnp.arange(OUT)
    mask = (ridx[:, None] == cidx[None, :]).astype(jnp.float32)
    asrc_w = att_src_pa.reshape(OUT)[:, None] * mask[:, :LANES]
    adst_w = att_dst_pa.reshape(OUT)[:, None] * mask[:, :LANES]
    bexp = (jnp.arange(LANES)[:, None] == ridx[None, :]).astype(jnp.float32)

    src = edge_index_pa[0]
    dst = edge_index_pa[1]

    nblk = 10
    rb = NA // nblk

    z_p, e_src, e_dst = pl.pallas_call(
        _proj_body,
        grid=(nblk,),
        in_specs=[
            pl.BlockSpec((rb, DIN), lambda i: (i, 0)),
            pl.BlockSpec((rb, DIN), lambda i: (i, 0)),
            pl.BlockSpec((DIN, OUT), lambda i: (0, 0)),
            pl.BlockSpec((DIN, OUT), lambda i: (0, 0)),
            pl.BlockSpec((1, OUT), lambda i: (0, 0)),
            pl.BlockSpec((1, OUT), lambda i: (0, 0)),
            pl.BlockSpec((OUT, LANES), lambda i: (0, 0)),
            pl.BlockSpec((OUT, LANES), lambda i: (0, 0)),
        ],
        out_specs=[
            pl.BlockSpec((rb, OUT), lambda i: (i, 0)),
            pl.BlockSpec((rb, LANES), lambda i: (i, 0)),
            pl.BlockSpec((rb, LANES), lambda i: (i, 0)),
        ],
        out_shape=[
            jax.ShapeDtypeStruct((NPA, OUT), jnp.float32),
            jax.ShapeDtypeStruct((NPA, LANES), jnp.float32),
            jax.ShapeDtypeStruct((NA, LANES), jnp.float32),
        ],
    )(x_author, x_paper, W_author, W_paper,
      b_author.reshape(1, OUT), b_paper.reshape(1, OUT), asrc_w, adst_w)

    zrow = jnp.zeros((zpt_last, OUT), jnp.float32)

    mesh = plsc.VectorSubcoreMesh(core_axis_name="c", subcore_axis_name="s",
                                  num_cores=NCORES, num_subcores=NSUB)

    NB = nchunk // 2

    @functools.partial(
        pl.kernel,
        out_type=[
            jax.ShapeDtypeStruct((NCORES * NA, OUT), jnp.float32),
            jax.ShapeDtypeStruct((NCORES * NA * LANES,), jnp.float32),
        ],
        mesh=mesh,
        compiler_params=pltpu.CompilerParams(use_tc_tiling_on_sc=False),
        scratch_types=[
            pltpu.VMEM((CH,), jnp.int32), pltpu.VMEM((CH,), jnp.int32),
            pltpu.VMEM((CH,), jnp.int32), pltpu.VMEM((CH,), jnp.int32),
            pltpu.VMEM((CH,), jnp.int32), pltpu.VMEM((CH,), jnp.int32),
            pltpu.VMEM((CH,), jnp.int32), pltpu.VMEM((CH,), jnp.int32),
            pltpu.VMEM((CH, LANES), jnp.float32), pltpu.VMEM((CH, LANES), jnp.float32),
            pltpu.VMEM((CH, OUT), jnp.float32), pltpu.VMEM((CH, OUT), jnp.float32),
            pltpu.VMEM((CH, LANES), jnp.float32), pltpu.VMEM((CH, LANES), jnp.float32),
            pltpu.VMEM((CH, OUT), jnp.float32), pltpu.VMEM((CH, OUT), jnp.float32),
            pltpu.VMEM((TRP * LANES,), jnp.float32),
            pltpu.VMEM_SHARED((RTOT8, OUT), jnp.float32),
            pltpu.SemaphoreType.DMA, pltpu.SemaphoreType.DMA,
            pltpu.SemaphoreType.DMA, pltpu.SemaphoreType.DMA,
            pltpu.SemaphoreType.DMA, pltpu.SemaphoreType.DMA,
        ],
    )
    def _edge_kernel(src_hbm, dst_hbm, zp_hbm, esrc_hbm, edst_hbm,
                     zrow_hbm, acc_out, den_out,
                     srcA, dstA, drowA, dscA, srcB, dstB, drowB, dscB,
                     asrcA, adstA, rowsA, exrowA,
                     asrcB, adstB, rowsB, exrowB,
                     den_tr, acc_sh,
                     semiA, semiB, semgA, semgB, semsA, semsB):
        c = lax.axis_index("c")
        s = lax.axis_index("s")
        wid = s * NCORES + c
        lanes_iota = lax.iota(jnp.int32, LANES)
        zeros16 = lanes_iota * 0
        zv = jnp.zeros((LANES,), jnp.float32)
        A = (srcA, dstA, drowA, asrcA, adstA, rowsA, exrowA, semiA, semgA,
             semsA, dscA)
        B = (srcB, dstB, drowB, asrcB, adstB, rowsB, exrowB, semiB, semgB,
             semsB, dscB)

        @pl.when(s < NSUB - 1)
        def _zero_main():
            pltpu.sync_copy(zrow_hbm.at[pl.ds(0, zpt)],
                            acc_sh.at[pl.ds(pl.multiple_of(s * zpt, 8), zpt)])

        @pl.when(s == NSUB - 1)
        def _zero_last():
            pltpu.sync_copy(
                zrow_hbm,
                acc_sh.at[pl.ds(pl.multiple_of((NSUB - 1) * zpt, 8), zpt_last)])

        def ebase(j):
            return pl.multiple_of(wid * epw + j * CH, 8)

        def start_idx(S, j):
            (src_v, dst_v, *_), semi = S[:3], S[7]
            b = ebase(j)
            pltpu.async_copy(src_hbm.at[pl.ds(b, CH)], src_v, semi)
            pltpu.async_copy(dst_hbm.at[pl.ds(b, CH)], dst_v, semi)

        def wait_idx(S):
            (src_v, dst_v), semi = S[:2], S[7]
            pltpu.make_async_copy(src_hbm.at[pl.ds(0, CH)], src_v, semi).wait()
            pltpu.make_async_copy(dst_hbm.at[pl.ds(0, CH)], dst_v, semi).wait()

        def start_gathers(S):
            (src_v, dst_v, _, asrc_v, adst_v, rows_v), semg = S[:6], S[8]
            pltpu.async_copy(esrc_hbm.at[src_v], asrc_v, semg)
            pltpu.async_copy(edst_hbm.at[dst_v], adst_v, semg)
            pltpu.async_copy(zp_hbm.at[src_v], rows_v, semg)

        def wait_gathers(S):
            (src_v, dst_v, _, asrc_v, adst_v, rows_v), semg = S[:6], S[8]
            pltpu.make_async_copy(esrc_hbm.at[src_v], asrc_v, semg).wait()
            pltpu.make_async_copy(edst_hbm.at[dst_v], adst_v, semg).wait()
            pltpu.make_async_copy(zp_hbm.at[src_v], rows_v, semg).wait()

        def start_scatters(S):
            (_, _, drow_v, _, _, rows_v, exrow_v), sems, dsc_v = S[:7], S[9], S[10]
            pltpu.async_copy(rows_v, acc_sh.at[dsc_v], sems, add=True)
            pltpu.async_copy(exrow_v, acc_sh.at[drow_v], sems, add=True)

        def wait_scatters(S):
            (_, _, drow_v, _, _, rows_v, exrow_v), sems, dsc_v = S[:7], S[9], S[10]
            pltpu.make_async_copy(rows_v, acc_sh.at[dsc_v], sems).wait()
            pltpu.make_async_copy(exrow_v, acc_sh.at[drow_v], sems).wait()

        def prezero(S):
            exrow_v = S[6]

            def zb(i, carry):
                for j2 in range(HEADS):
                    exrow_v[i, pl.ds(j2 * LANES, LANES)] = zv
                return carry

            lax.fori_loop(0, CH, zb, 0)

        def compute(S):
            (_, dst_v, drow_v, asrc_v, adst_v, rows_v, exrow_v) = S[:7]
            dsc_v = S[10]
            for g, ks in ((0, range(LANES)), (16, range(LANES)),
                          (24, range(8, LANES))):
                dvec = dst_v[pl.ds(g, LANES)]
                dsc_v[pl.ds(g, LANES)] = dvec
                drow_v[pl.ds(g, LANES)] = NA + (dvec >> 4)
                for k in ks:
                    e = g + k
                    va = asrc_v[e, pl.ds(0, LANES)] + adst_v[e, pl.ds(0, LANES)]
                    al = jnp.where(va >= 0.0, va, 0.2 * va)
                    exv = jnp.exp(al)
                    dk = dvec[k]
                    q16 = pl.multiple_of(((dk & (LANES - 1)) >> 1) * LANES,
                                         LANES)
                    r8 = (dk & 1) * HEADS
                    shifted = exv.at[(lanes_iota - r8) & (LANES - 1)].get(
                        mode="promise_in_bounds")
                    vmask = (lanes_iota >= r8) & (lanes_iota < r8 + HEADS)
                    exrow_v[e, pl.ds(q16, LANES)] = jnp.where(vmask, shifted, 0.0)
                    for h in range(HEADS):
                        r = rows_v[e, pl.ds(h * DH, DH)]
                        rows_v[e, pl.ds(h * DH, DH)] = r * _lane_bcast(
                            exv, zeros16, h)

        plsc.subcore_barrier()

        b0 = ebase(0)
        pltpu.sync_copy(src_hbm.at[pl.ds(b0, CH)], srcA)
        pltpu.sync_copy(dst_hbm.at[pl.ds(b0, CH)], dstA)
        start_gathers(A)
        prezero(A)
        b1 = ebase(1)
        pltpu.sync_copy(src_hbm.at[pl.ds(b1, CH)], srcB)
        pltpu.sync_copy(dst_hbm.at[pl.ds(b1, CH)], dstB)
        start_gathers(B)
        prezero(B)

        def body(j2, carry):
            j = 2 * j2
            more = j2 < NB - 1
            wait_gathers(A)
            compute(A)

            @pl.when(more)
            def _pi_a():
                start_idx(A, j + 2)

            start_scatters(A)
            wait_gathers(B)
            compute(B)

            @pl.when(more)
            def _pi_b():
                start_idx(B, j + 3)

            start_scatters(B)

            @pl.when(more)
            def _prefetch():
                wait_scatters(A)
                wait_idx(A)
                start_gathers(A)
                prezero(A)
                wait_scatters(B)
                wait_idx(B)
                start_gathers(B)
                prezero(B)

            return carry

        lax.fori_loop(0, NB, body, 0)
        wait_scatters(A)
        wait_scatters(B)
        plsc.subcore_barrier()

        pltpu.sync_copy(acc_sh.at[pl.ds(NA + s * drpt, CH)], exrowA)

        def make_tr(p0):
            def tr_body(nloc, carry):
                n2 = p0 + nloc
                row = n2 >> 4
                q16 = pl.multiple_of(((n2 & (LANES - 1)) >> 1) * LANES, LANES)
                r8 = (n2 & 1) * HEADS
                v = exrowA[row, pl.ds(q16, LANES)]
                den_tr[pl.ds(pl.multiple_of(nloc * LANES, LANES), LANES)] = v.at[
                    (lanes_iota + r8) & (LANES - 1)].get(mode="promise_in_bounds")
                return carry
            return tr_body

        out_node0 = pl.multiple_of(c * NA + s * rpt, 8)
        for p in range(ntr):
            lax.fori_loop(0, TRP, make_tr(p * TRP), 0)
            pltpu.sync_copy(
                den_tr,
                den_out.at[pl.ds((out_node0 + p * TRP) * LANES, TRP * LANES)])

        @pl.when(s == NSUB - 1)
        def _tr_tail():
            tail = rpt_last - rpt
            lax.fori_loop(0, tail, make_tr(ntr * TRP), 0)
            pltpu.sync_copy(
                den_tr.at[pl.ds(0, tail * LANES)],
                den_out.at[pl.ds((out_node0 + ntr * TRP) * LANES, tail * LANES)])

        @pl.when(s < NSUB - 1)
        def _out_main():
            pltpu.sync_copy(acc_sh.at[pl.ds(pl.multiple_of(s * rpt, 8), rpt)],
                            acc_out.at[pl.ds(out_node0, rpt)])

        @pl.when(s == NSUB - 1)
        def _out_last():
            pltpu.sync_copy(
                acc_sh.at[pl.ds(pl.multiple_of((NSUB - 1) * rpt, 8), rpt_last)],
                acc_out.at[pl.ds(out_node0, rpt_last)])

    acc, den_flat = _edge_kernel(src, dst, z_p, e_src, e_dst, zrow)
    den = den_flat.reshape(NCORES * NA, LANES)

    out = pl.pallas_call(
        _final_body,
        grid=(nblk,),
        in_specs=[
            pl.BlockSpec((rb, OUT), lambda i: (i, 0)),
            pl.BlockSpec((rb, OUT), lambda i: (i + nblk, 0)),
            pl.BlockSpec((rb, LANES), lambda i: (i, 0)),
            pl.BlockSpec((rb, LANES), lambda i: (i + nblk, 0)),
            pl.BlockSpec((LANES, OUT), lambda i: (0, 0)),
            pl.BlockSpec((OUT, DOUT), lambda i: (0, 0)),
            pl.BlockSpec((1, DOUT), lambda i: (0, 0)),
        ],
        out_specs=pl.BlockSpec((rb, DOUT), lambda i: (i, 0)),
        out_shape=jax.ShapeDtypeStruct((NA, DOUT), jnp.float32),
    )(acc, acc, den, den, bexp, W_out, b_out.reshape(1, DOUT))
    return out


# speedup vs baseline: 1.8518x; 1.8518x over previous
import functools

import jax
import jax.numpy as jnp
from jax import lax
from jax.experimental import pallas as pl
from jax.experimental.pallas import tpu as pltpu
from jax.experimental.pallas import tpu_sc as plsc

HEADS = 8
DH = 16
LANES = 16
NCORES = 2
NSUB = 16
NW = NCORES * NSUB
CH = 40
TRP = 104


def _lane_bcast(v, zeros16, lane):
    return v.at[zeros16 + lane].get(mode="promise_in_bounds")


def _proj_body(xa_ref, xp_ref, wa_ref, wp_ref, ba_ref, bp_ref,
               asrc_w_ref, adst_w_ref, zp_ref, esrc_ref, edst_ref):
    zp = jnp.dot(xp_ref[...], wp_ref[...], preferred_element_type=jnp.float32)
    zp = zp + bp_ref[...]
    za = jnp.dot(xa_ref[...], wa_ref[...], preferred_element_type=jnp.float32)
    za = za + ba_ref[...]
    zp_ref[...] = zp
    esrc_ref[...] = jnp.dot(zp, asrc_w_ref[...], preferred_element_type=jnp.float32)
    edst_ref[...] = jnp.dot(za, adst_w_ref[...], preferred_element_type=jnp.float32)


def _final_body(acc0_ref, acc1_ref, den0_ref, den1_ref, bexp_ref, wout_ref,
                bout_ref, out_ref):
    a = acc0_ref[...] + acc1_ref[...]
    dsum = den0_ref[...] + den1_ref[...] + 1e-16
    dinv = 1.0 / dsum
    expand = jnp.dot(dinv, bexp_ref[...], preferred_element_type=jnp.float32)
    h = jnp.maximum(a * expand, 0.0)
    out_ref[...] = jnp.dot(h, wout_ref[...], preferred_element_type=jnp.float32) + bout_ref[...]


def kernel(x_author, x_paper, edge_index_ap, edge_index_pa,
           W_author, b_author, W_paper, b_paper,
           att_src_ap, att_dst_ap, att_src_pa, att_dst_pa,
           Wk, bk, q, W_out, b_out):
    NA = x_author.shape[0]
    NPA = x_paper.shape[0]
    DIN = x_author.shape[1]
    OUT = W_author.shape[1]
    DOUT = W_out.shape[1]
    E = edge_index_pa.shape[1]

    assert OUT == HEADS * DH and E % NW == 0
    assert NA % LANES == 0
    epw = E // NW
    assert epw % CH == 0
    nchunk = epw // CH
    assert nchunk % 2 == 0
    rpt = NA // NSUB - (NA // NSUB) % LANES
    rpt_last = NA - (NSUB - 1) * rpt
    drpt = rpt // LANES
    DTOT = NA // LANES
    RTOT = NA + DTOT
    RTOT8 = -(-RTOT // 8) * 8
    zpt = (RTOT8 // NSUB) // 8 * 8
    zpt_last = RTOT8 - (NSUB - 1) * zpt
    ntr = rpt // TRP
    assert ntr * TRP == rpt and rpt_last - rpt <= TRP

    ridx = jnp.arange(OUT) // DH
    cidx = jnp.arange(OUT)
    mask = (ridx[:, None] == cidx[None, :]).astype(jnp.float32)
    asrc_w = att_src_pa.reshape(OUT)[:, None] * mask[:, :LANES]
    adst_w = att_dst_pa.reshape(OUT)[:, None] * mask[:, :LANES]
    bexp = (jnp.arange(LANES)[:, None] == ridx[None, :]).astype(jnp.float32)

    src = edge_index_pa[0]
    dst = edge_index_pa[1]

    nblk = 10
    rb = NA // nblk

    z_p, e_src, e_dst = pl.pallas_call(
        _proj_body,
        grid=(nblk,),
        in_specs=[
            pl.BlockSpec((rb, DIN), lambda i: (i, 0)),
            pl.BlockSpec((rb, DIN), lambda i: (i, 0)),
            pl.BlockSpec((DIN, OUT), lambda i: (0, 0)),
            pl.BlockSpec((DIN, OUT), lambda i: (0, 0)),
            pl.BlockSpec((1, OUT), lambda i: (0, 0)),
            pl.BlockSpec((1, OUT), lambda i: (0, 0)),
            pl.BlockSpec((OUT, LANES), lambda i: (0, 0)),
            pl.BlockSpec((OUT, LANES), lambda i: (0, 0)),
        ],
        out_specs=[
            pl.BlockSpec((rb, OUT), lambda i: (i, 0)),
            pl.BlockSpec((rb, LANES), lambda i: (i, 0)),
            pl.BlockSpec((rb, LANES), lambda i: (i, 0)),
        ],
        out_shape=[
            jax.ShapeDtypeStruct((NPA, OUT), jnp.float32),
            jax.ShapeDtypeStruct((NPA, LANES), jnp.float32),
            jax.ShapeDtypeStruct((NA, LANES), jnp.float32),
        ],
    )(x_author, x_paper, W_author, W_paper,
      b_author.reshape(1, OUT), b_paper.reshape(1, OUT), asrc_w, adst_w)

    zrow = jnp.zeros((rpt_last, OUT), jnp.float32)
    zrow16 = jnp.zeros((rpt_last, LANES), jnp.float32)

    mesh = plsc.VectorSubcoreMesh(core_axis_name="c", subcore_axis_name="s",
                                  num_cores=NCORES, num_subcores=NSUB)

    NB = nchunk // 2

    @functools.partial(
        pl.kernel,
        out_type=[
            jax.ShapeDtypeStruct((NCORES * NA, OUT), jnp.float32),
            jax.ShapeDtypeStruct((NCORES * NA, LANES), jnp.float32),
        ],
        mesh=mesh,
        compiler_params=pltpu.CompilerParams(use_tc_tiling_on_sc=False),
        scratch_types=[
            pltpu.VMEM((CH,), jnp.int32), pltpu.VMEM((CH,), jnp.int32),
            pltpu.VMEM((CH,), jnp.int32),
            pltpu.VMEM((CH,), jnp.int32), pltpu.VMEM((CH,), jnp.int32),
            pltpu.VMEM((CH,), jnp.int32),
            pltpu.VMEM((CH, LANES), jnp.float32), pltpu.VMEM((CH, LANES), jnp.float32),
            pltpu.VMEM((CH, OUT), jnp.float32), pltpu.VMEM((CH, LANES), jnp.float32),
            pltpu.VMEM((CH, LANES), jnp.float32), pltpu.VMEM((CH, LANES), jnp.float32),
            pltpu.VMEM((CH, OUT), jnp.float32), pltpu.VMEM((CH, LANES), jnp.float32),
            pltpu.VMEM_SHARED((NA, OUT), jnp.float32),
            pltpu.VMEM_SHARED((NA, LANES), jnp.float32),
            pltpu.SemaphoreType.DMA, pltpu.SemaphoreType.DMA,
            pltpu.SemaphoreType.DMA, pltpu.SemaphoreType.DMA,
            pltpu.SemaphoreType.DMA, pltpu.SemaphoreType.DMA,
        ],
    )
    def _edge_kernel(src_hbm, dst_hbm, zp_hbm, esrc_hbm, edst_hbm,
                     zrow_hbm, zrow16_hbm, acc_out, den_out,
                     srcA, dstA, dscA, srcB, dstB, dscB,
                     asrcA, adstA, rowsA, exnA,
                     asrcB, adstB, rowsB, exnB,
                     acc_sh, den_sh,
                     semiA, semiB, semgA, semgB, semsA, semsB):
        c = lax.axis_index("c")
        s = lax.axis_index("s")
        wid = s * NCORES + c
        lanes_iota = lax.iota(jnp.int32, LANES)
        zeros16 = lanes_iota * 0
        zv = jnp.zeros((LANES,), jnp.float32)
        A = (srcA, dstA, None, asrcA, adstA, rowsA, exnA, semiA, semgA,
             semsA, dscA)
        B = (srcB, dstB, None, asrcB, adstB, rowsB, exnB, semiB, semgB,
             semsB, dscB)

        @pl.when(s < NSUB - 1)
        def _zero_main():
            r0 = pl.multiple_of(s * rpt, 8)
            pltpu.sync_copy(zrow_hbm.at[pl.ds(0, rpt)], acc_sh.at[pl.ds(r0, rpt)])
            pltpu.sync_copy(zrow16_hbm.at[pl.ds(0, rpt)],
                            den_sh.at[pl.ds(r0, rpt)])

        @pl.when(s == NSUB - 1)
        def _zero_last():
            r0 = pl.multiple_of((NSUB - 1) * rpt, 8)
            pltpu.sync_copy(zrow_hbm.at[pl.ds(0, rpt_last)],
                            acc_sh.at[pl.ds(r0, rpt_last)])
            pltpu.sync_copy(zrow16_hbm.at[pl.ds(0, rpt_last)],
                            den_sh.at[pl.ds(r0, rpt_last)])

        def ebase(j):
            return pl.multiple_of(wid * epw + j * CH, 8)

        def start_idx(S, j):
            (src_v, dst_v, *_), semi = S[:3], S[7]
            b = ebase(j)
            pltpu.async_copy(src_hbm.at[pl.ds(b, CH)], src_v, semi)
            pltpu.async_copy(dst_hbm.at[pl.ds(b, CH)], dst_v, semi)

        def wait_idx(S):
            (src_v, dst_v), semi = S[:2], S[7]
            pltpu.make_async_copy(src_hbm.at[pl.ds(0, CH)], src_v, semi).wait()
            pltpu.make_async_copy(dst_hbm.at[pl.ds(0, CH)], dst_v, semi).wait()

        def start_gathers(S):
            (src_v, dst_v, _, asrc_v, adst_v, rows_v), semg = S[:6], S[8]
            pltpu.async_copy(esrc_hbm.at[src_v], asrc_v, semg)
            pltpu.async_copy(edst_hbm.at[dst_v], adst_v, semg)
            pltpu.async_copy(zp_hbm.at[src_v], rows_v, semg)

        def wait_gathers(S):
            (src_v, dst_v, _, asrc_v, adst_v, rows_v), semg = S[:6], S[8]
            pltpu.make_async_copy(esrc_hbm.at[src_v], asrc_v, semg).wait()
            pltpu.make_async_copy(edst_hbm.at[dst_v], adst_v, semg).wait()
            pltpu.make_async_copy(zp_hbm.at[src_v], rows_v, semg).wait()

        def start_scatters(S):
            (_, _, _, _, _, rows_v, exn_v), sems, dsc_v = S[:7], S[9], S[10]
            pltpu.async_copy(rows_v, acc_sh.at[dsc_v], sems, add=True)
            pltpu.async_copy(exn_v, den_sh.at[dsc_v], sems, add=True)

        def wait_scatters(S):
            (_, _, _, _, _, rows_v, exn_v), sems, dsc_v = S[:7], S[9], S[10]
            pltpu.make_async_copy(rows_v, acc_sh.at[dsc_v], sems).wait()
            pltpu.make_async_copy(exn_v, den_sh.at[dsc_v], sems).wait()

        lane_mask = lanes_iota < HEADS

        def compute(S):
            (_, dst_v, _, asrc_v, adst_v, rows_v, exn_v) = S[:7]
            dsc_v = S[10]
            for g in (0, 16, 24):
                dvec = dst_v[pl.ds(g, LANES)]
                dsc_v[pl.ds(g, LANES)] = dvec
            for e in range(CH):
                va = asrc_v[e, pl.ds(0, LANES)] + adst_v[e, pl.ds(0, LANES)]
                al = jnp.where(va >= 0.0, va, 0.2 * va)
                exv = jnp.exp(al)
                exn_v[e, pl.ds(0, LANES)] = jnp.where(lane_mask, exv, 0.0)
                for h in range(HEADS):
                    r = rows_v[e, pl.ds(h * DH, DH)]
                    rows_v[e, pl.ds(h * DH, DH)] = r * _lane_bcast(
                        exv, zeros16, h)

        plsc.subcore_barrier()

        b0 = ebase(0)
        pltpu.sync_copy(src_hbm.at[pl.ds(b0, CH)], srcA)
        pltpu.sync_copy(dst_hbm.at[pl.ds(b0, CH)], dstA)
        start_gathers(A)
        b1 = ebase(1)
        pltpu.sync_copy(src_hbm.at[pl.ds(b1, CH)], srcB)
        pltpu.sync_copy(dst_hbm.at[pl.ds(b1, CH)], dstB)
        start_gathers(B)

        def body(j2, carry):
            j = 2 * j2
            more = j2 < NB - 1
            wait_gathers(A)
            compute(A)

            @pl.when(more)
            def _pi_a():
                start_idx(A, j + 2)

            start_scatters(A)
            wait_gathers(B)

            @pl.when(more)
            def _pg_a():
                wait_scatters(A)
                wait_idx(A)
                start_gathers(A)

            compute(B)

            @pl.when(more)
            def _pi_b():
                start_idx(B, j + 3)

            start_scatters(B)

            @pl.when(more)
            def _pg_b():
                wait_scatters(B)
                wait_idx(B)
                start_gathers(B)

            return carry

        lax.fori_loop(0, NB, body, 0)
        wait_scatters(A)
        wait_scatters(B)
        plsc.subcore_barrier()

        out_node0 = pl.multiple_of(c * NA + s * rpt, 8)

        @pl.when(s < NSUB - 1)
        def _out_main():
            r0 = pl.multiple_of(s * rpt, 8)
            pltpu.sync_copy(acc_sh.at[pl.ds(r0, rpt)],
                            acc_out.at[pl.ds(out_node0, rpt)])
            pltpu.sync_copy(den_sh.at[pl.ds(r0, rpt)],
                            den_out.at[pl.ds(out_node0, rpt)])

        @pl.when(s == NSUB - 1)
        def _out_last():
            r0 = pl.multiple_of((NSUB - 1) * rpt, 8)
            pltpu.sync_copy(acc_sh.at[pl.ds(r0, rpt_last)],
                            acc_out.at[pl.ds(out_node0, rpt_last)])
            pltpu.sync_copy(den_sh.at[pl.ds(r0, rpt_last)],
                            den_out.at[pl.ds(out_node0, rpt_last)])

    acc, den = _edge_kernel(src, dst, z_p, e_src, e_dst, zrow, zrow16)

    out = pl.pallas_call(
        _final_body,
        grid=(nblk,),
        in_specs=[
            pl.BlockSpec((rb, OUT), lambda i: (i, 0)),
            pl.BlockSpec((rb, OUT), lambda i: (i + nblk, 0)),
            pl.BlockSpec((rb, LANES), lambda i: (i, 0)),
            pl.BlockSpec((rb, LANES), lambda i: (i + nblk, 0)),
            pl.BlockSpec((LANES, OUT), lambda i: (0, 0)),
            pl.BlockSpec((OUT, DOUT), lambda i: (0, 0)),
            pl.BlockSpec((1, DOUT), lambda i: (0, 0)),
        ],
        out_specs=pl.BlockSpec((rb, DOUT), lambda i: (i, 0)),
        out_shape=jax.ShapeDtypeStruct((NA, DOUT), jnp.float32),
    )(acc, acc, den, den, bexp, W_out, b_out.reshape(1, DOUT))
    return out


# speedup vs baseline: 1.8724x; 1.0111x over previous
import functools

import jax
import jax.numpy as jnp
from jax import lax
from jax.experimental import pallas as pl
from jax.experimental.pallas import tpu as pltpu
from jax.experimental.pallas import tpu_sc as plsc

HEADS = 8
DH = 16
LANES = 16
NCORES = 2
NSUB = 16
NW = NCORES * NSUB
CH = 40
TRP = 104


def _lane_bcast(v, zeros16, lane):
    return v.at[zeros16 + lane].get(mode="promise_in_bounds")


def _proj_body(xa_ref, xp_ref, wa_ref, wp_ref, ba_ref, bp_ref,
               asrc_w_ref, adst_w_ref, zp_ref, esrc_ref, edst_ref):
    zp = jnp.dot(xp_ref[...], wp_ref[...], preferred_element_type=jnp.float32)
    zp = zp + bp_ref[...]
    za = jnp.dot(xa_ref[...], wa_ref[...], preferred_element_type=jnp.float32)
    za = za + ba_ref[...]
    zp_ref[...] = zp
    esrc_ref[...] = jnp.dot(zp, asrc_w_ref[...], preferred_element_type=jnp.float32)
    edst_ref[...] = jnp.dot(za, adst_w_ref[...], preferred_element_type=jnp.float32)


def _final_body(acc0_ref, acc1_ref, den0_ref, den1_ref, bexp_ref, wout_ref,
                bout_ref, out_ref):
    a = acc0_ref[...] + acc1_ref[...]
    dsum = den0_ref[...] + den1_ref[...] + 1e-16
    dinv = 1.0 / dsum
    expand = jnp.dot(dinv, bexp_ref[...], preferred_element_type=jnp.float32)
    h = jnp.maximum(a * expand, 0.0)
    out_ref[...] = jnp.dot(h, wout_ref[...], preferred_element_type=jnp.float32) + bout_ref[...]


def kernel(x_author, x_paper, edge_index_ap, edge_index_pa,
           W_author, b_author, W_paper, b_paper,
           att_src_ap, att_dst_ap, att_src_pa, att_dst_pa,
           Wk, bk, q, W_out, b_out):
    NA = x_author.shape[0]
    NPA = x_paper.shape[0]
    DIN = x_author.shape[1]
    OUT = W_author.shape[1]
    DOUT = W_out.shape[1]
    E = edge_index_pa.shape[1]

    assert OUT == HEADS * DH and E % NW == 0
    assert NA % LANES == 0
    epw = E // NW
    assert epw % CH == 0
    nchunk = epw // CH
    assert nchunk % 2 == 0
    rpt = NA // NSUB - (NA // NSUB) % LANES
    rpt_last = NA - (NSUB - 1) * rpt
    drpt = rpt // LANES
    DTOT = NA // LANES
    RTOT = NA + DTOT
    RTOT8 = -(-RTOT // 8) * 8
    zpt = (RTOT8 // NSUB) // 8 * 8
    zpt_last = RTOT8 - (NSUB - 1) * zpt
    ntr = rpt // TRP
    assert ntr * TRP == rpt and rpt_last - rpt <= TRP

    ridx = jnp.arange(OUT) // DH
    cidx = jnp.arange(OUT)
    mask = (ridx[:, None] == cidx[None, :]).astype(jnp.float32)
    asrc_w = att_src_pa.reshape(OUT)[:, None] * mask[:, :LANES]
    adst_w = att_dst_pa.reshape(OUT)[:, None] * mask[:, :LANES]
    bexp = (jnp.arange(LANES)[:, None] == ridx[None, :]).astype(jnp.float32)

    src = edge_index_pa[0]
    dst = edge_index_pa[1]

    nblk = 10
    rb = NA // nblk

    z_p, e_src, e_dst = pl.pallas_call(
        _proj_body,
        grid=(nblk,),
        in_specs=[
            pl.BlockSpec((rb, DIN), lambda i: (i, 0)),
            pl.BlockSpec((rb, DIN), lambda i: (i, 0)),
            pl.BlockSpec((DIN, OUT), lambda i: (0, 0)),
            pl.BlockSpec((DIN, OUT), lambda i: (0, 0)),
            pl.BlockSpec((1, OUT), lambda i: (0, 0)),
            pl.BlockSpec((1, OUT), lambda i: (0, 0)),
            pl.BlockSpec((OUT, LANES), lambda i: (0, 0)),
            pl.BlockSpec((OUT, LANES), lambda i: (0, 0)),
        ],
        out_specs=[
            pl.BlockSpec((rb, OUT), lambda i: (i, 0)),
            pl.BlockSpec((rb, LANES), lambda i: (i, 0)),
            pl.BlockSpec((rb, LANES), lambda i: (i, 0)),
        ],
        out_shape=[
            jax.ShapeDtypeStruct((NPA, OUT), jnp.float32),
            jax.ShapeDtypeStruct((NPA, LANES), jnp.float32),
            jax.ShapeDtypeStruct((NA, LANES), jnp.float32),
        ],
    )(x_author, x_paper, W_author, W_paper,
      b_author.reshape(1, OUT), b_paper.reshape(1, OUT), asrc_w, adst_w)

    zrow = jnp.zeros((rpt_last, OUT), jnp.float32)
    zrow16 = jnp.zeros((rpt_last, LANES), jnp.float32)

    mesh = plsc.VectorSubcoreMesh(core_axis_name="c", subcore_axis_name="s",
                                  num_cores=NCORES, num_subcores=NSUB)

    NB = nchunk // 2

    @functools.partial(
        pl.kernel,
        out_type=[
            jax.ShapeDtypeStruct((NCORES * NA, OUT), jnp.float32),
            jax.ShapeDtypeStruct((NCORES * NA, LANES), jnp.float32),
        ],
        mesh=mesh,
        compiler_params=pltpu.CompilerParams(use_tc_tiling_on_sc=False),
        scratch_types=[
            pltpu.VMEM((CH,), jnp.int32), pltpu.VMEM((CH,), jnp.int32),
            pltpu.VMEM((CH,), jnp.int32),
            pltpu.VMEM((CH,), jnp.int32), pltpu.VMEM((CH,), jnp.int32),
            pltpu.VMEM((CH,), jnp.int32),
            pltpu.VMEM((CH, LANES), jnp.float32), pltpu.VMEM((CH, LANES), jnp.float32),
            pltpu.VMEM((CH, OUT), jnp.float32), pltpu.VMEM((CH, LANES), jnp.float32),
            pltpu.VMEM((CH, LANES), jnp.float32), pltpu.VMEM((CH, LANES), jnp.float32),
            pltpu.VMEM((CH, OUT), jnp.float32), pltpu.VMEM((CH, LANES), jnp.float32),
            pltpu.VMEM_SHARED((NA, OUT), jnp.float32),
            pltpu.VMEM_SHARED((NA, LANES), jnp.float32),
            pltpu.SemaphoreType.DMA, pltpu.SemaphoreType.DMA,
            pltpu.SemaphoreType.DMA, pltpu.SemaphoreType.DMA,
            pltpu.SemaphoreType.DMA, pltpu.SemaphoreType.DMA,
        ],
    )
    def _edge_kernel(src_hbm, dst_hbm, zp_hbm, esrc_hbm, edst_hbm,
                     zrow_hbm, zrow16_hbm, acc_out, den_out,
                     srcA, dstA, dscA, srcB, dstB, dscB,
                     asrcA, adstA, rowsA, exnA,
                     asrcB, adstB, rowsB, exnB,
                     acc_sh, den_sh,
                     semiA, semiB, semgA, semgB, semsA, semsB):
        c = lax.axis_index("c")
        s = lax.axis_index("s")
        wid = s * NCORES + c
        lanes_iota = lax.iota(jnp.int32, LANES)
        zeros16 = lanes_iota * 0
        zv = jnp.zeros((LANES,), jnp.float32)
        A = (srcA, dstA, None, asrcA, adstA, rowsA, exnA, semiA, semgA,
             semsA, dscA)
        B = (srcB, dstB, None, asrcB, adstB, rowsB, exnB, semiB, semgB,
             semsB, dscB)

        @pl.when(s < NSUB - 1)
        def _zero_main():
            r0 = pl.multiple_of(s * rpt, 8)
            pltpu.sync_copy(zrow_hbm.at[pl.ds(0, rpt)], acc_sh.at[pl.ds(r0, rpt)])
            pltpu.sync_copy(zrow16_hbm.at[pl.ds(0, rpt)],
                            den_sh.at[pl.ds(r0, rpt)])

        @pl.when(s == NSUB - 1)
        def _zero_last():
            r0 = pl.multiple_of((NSUB - 1) * rpt, 8)
            pltpu.sync_copy(zrow_hbm.at[pl.ds(0, rpt_last)],
                            acc_sh.at[pl.ds(r0, rpt_last)])
            pltpu.sync_copy(zrow16_hbm.at[pl.ds(0, rpt_last)],
                            den_sh.at[pl.ds(r0, rpt_last)])

        def ebase(j):
            return pl.multiple_of(wid * epw + j * CH, 8)

        def start_idx(S, j):
            (src_v, dst_v, *_), semi = S[:3], S[7]
            b = ebase(j)
            pltpu.async_copy(src_hbm.at[pl.ds(b, CH)], src_v, semi)
            pltpu.async_copy(dst_hbm.at[pl.ds(b, CH)], dst_v, semi)

        def wait_idx(S):
            (src_v, dst_v), semi = S[:2], S[7]
            pltpu.make_async_copy(src_hbm.at[pl.ds(0, CH)], src_v, semi).wait()
            pltpu.make_async_copy(dst_hbm.at[pl.ds(0, CH)], dst_v, semi).wait()

        def start_gathers(S):
            (src_v, dst_v, _, asrc_v, adst_v, rows_v), semg = S[:6], S[8]
            pltpu.async_copy(esrc_hbm.at[src_v], asrc_v, semg)
            pltpu.async_copy(edst_hbm.at[dst_v], adst_v, semg)
            pltpu.async_copy(zp_hbm.at[src_v], rows_v, semg)

        def wait_gathers(S):
            (src_v, dst_v, _, asrc_v, adst_v, rows_v), semg = S[:6], S[8]
            pltpu.make_async_copy(esrc_hbm.at[src_v], asrc_v, semg).wait()
            pltpu.make_async_copy(edst_hbm.at[dst_v], adst_v, semg).wait()
            pltpu.make_async_copy(zp_hbm.at[src_v], rows_v, semg).wait()

        def start_scatters(S):
            (_, _, _, _, _, rows_v, exn_v), sems, dsc_v = S[:7], S[9], S[10]
            pltpu.async_copy(rows_v, acc_sh.at[dsc_v], sems, add=True)
            pltpu.async_copy(exn_v, den_sh.at[dsc_v], sems, add=True)

        def wait_scatters(S):
            (_, _, _, _, _, rows_v, exn_v), sems, dsc_v = S[:7], S[9], S[10]
            pltpu.make_async_copy(rows_v, acc_sh.at[dsc_v], sems).wait()
            pltpu.make_async_copy(exn_v, den_sh.at[dsc_v], sems).wait()

        lane_mask = lanes_iota < HEADS

        def compute(S):
            (_, dst_v, _, asrc_v, adst_v, rows_v, exn_v) = S[:7]
            dsc_v = S[10]
            for g in (0, 16, 24):
                dvec = dst_v[pl.ds(g, LANES)]
                dsc_v[pl.ds(g, LANES)] = dvec
            for e in range(CH):
                va = asrc_v[e, pl.ds(0, LANES)] + adst_v[e, pl.ds(0, LANES)]
                al = jnp.where(va >= 0.0, va, 0.2 * va)
                exv = jnp.exp(al)
                exn_v[e, pl.ds(0, LANES)] = jnp.where(lane_mask, exv, 0.0)
                for h in range(HEADS):
                    r = rows_v[e, pl.ds(h * DH, DH)]
                    rows_v[e, pl.ds(h * DH, DH)] = r * _lane_bcast(
                        exv, zeros16, h)

        plsc.subcore_barrier()

        b0 = ebase(0)
        pltpu.sync_copy(src_hbm.at[pl.ds(b0, CH)], srcA)
        pltpu.sync_copy(dst_hbm.at[pl.ds(b0, CH)], dstA)
        start_gathers(A)
        b1 = ebase(1)
        pltpu.sync_copy(src_hbm.at[pl.ds(b1, CH)], srcB)
        pltpu.sync_copy(dst_hbm.at[pl.ds(b1, CH)], dstB)
        start_gathers(B)

        def body(j2, carry):
            j = 2 * j2
            more = j2 < NB - 1
            wait_gathers(A)
            compute(A)

            @pl.when(more)
            def _pi_a():
                start_idx(A, j + 2)

            start_scatters(A)
            wait_gathers(B)
            compute(B)

            @pl.when(more)
            def _pi_b():
                start_idx(B, j + 3)

            start_scatters(B)

            @pl.when(more)
            def _prefetch():
                wait_scatters(A)
                wait_idx(A)
                start_gathers(A)
                wait_scatters(B)
                wait_idx(B)
                start_gathers(B)

            return carry

        lax.fori_loop(0, NB, body, 0)
        wait_scatters(A)
        wait_scatters(B)
        plsc.subcore_barrier()

        out_node0 = pl.multiple_of(c * NA + s * rpt, 8)

        @pl.when(s < NSUB - 1)
        def _out_main():
            r0 = pl.multiple_of(s * rpt, 8)
            pltpu.sync_copy(acc_sh.at[pl.ds(r0, rpt)],
                            acc_out.at[pl.ds(out_node0, rpt)])
            pltpu.sync_copy(den_sh.at[pl.ds(r0, rpt)],
                            den_out.at[pl.ds(out_node0, rpt)])

        @pl.when(s == NSUB - 1)
        def _out_last():
            r0 = pl.multiple_of((NSUB - 1) * rpt, 8)
            pltpu.sync_copy(acc_sh.at[pl.ds(r0, rpt_last)],
                            acc_out.at[pl.ds(out_node0, rpt_last)])
            pltpu.sync_copy(den_sh.at[pl.ds(r0, rpt_last)],
                            den_out.at[pl.ds(out_node0, rpt_last)])

    acc, den = _edge_kernel(src, dst, z_p, e_src, e_dst, zrow, zrow16)

    out = pl.pallas_call(
        _final_body,
        grid=(nblk,),
        in_specs=[
            pl.BlockSpec((rb, OUT), lambda i: (i, 0)),
            pl.BlockSpec((rb, OUT), lambda i: (i + nblk, 0)),
            pl.BlockSpec((rb, LANES), lambda i: (i, 0)),
            pl.BlockSpec((rb, LANES), lambda i: (i + nblk, 0)),
            pl.BlockSpec((LANES, OUT), lambda i: (0, 0)),
            pl.BlockSpec((OUT, DOUT), lambda i: (0, 0)),
            pl.BlockSpec((1, DOUT), lambda i: (0, 0)),
        ],
        out_specs=pl.BlockSpec((rb, DOUT), lambda i: (i, 0)),
        out_shape=jax.ShapeDtypeStruct((NA, DOUT), jnp.float32),
    )(acc, acc, den, den, bexp, W_out, b_out.reshape(1, DOUT))
    return out
